# baseline (speedup 1.0000x reference)
import functools

import numpy as np
import jax
import jax.numpy as jnp
from jax import lax
from jax.experimental import pallas as pl
from jax.experimental.pallas import tpu as pltpu
from jax.experimental.pallas import tpu_sc as plsc


def _popcount_py(x):
    return bin(x).count("1")


def _reorder_sign_py(a, b):
    a >>= 1
    s = 0
    while a:
        s += _popcount_py(a & b)
        a >>= 1
    return 1.0 if s % 2 == 0 else -1.0


_GRADES = np.array([_popcount_py(i) for i in range(32)])
_SIGN_K = np.array(
    [[_reorder_sign_py(k ^ b, b) for b in range(32)] for k in range(32)],
    dtype=np.float32)
_VEC_IDX = np.array([1, 2, 4, 8, 16])

_GRAY = [t ^ (t >> 1) for t in range(32)]
_GRAY_FLIP = [int(np.log2(_GRAY[t] ^ _GRAY[t - 1])) for t in range(1, 32)]

_GM = np.stack([( _GRADES == g).astype(np.float32) for g in range(6)])
_SG = (_GRADES[:, None] == _GRADES[None, :]).astype(np.float32)
_SGB = np.kron(np.eye(8, dtype=np.float32), _SG)
_BITM = np.stack([((np.arange(32) >> j) & 1).astype(np.float32)
                  for j in range(5)])
_SGN_ROWS = _SIGN_K.T.copy()

_HIDDEN = 8
_N_LAYERS = 3
_BE = 640


def _mm(a, b):
    return jax.lax.dot_general(
        a, b, (((1,), (0,)), ((), ())),
        precision=jax.lax.Precision.HIGHEST,
        preferred_element_type=jnp.float32)


def _gp1(xc, zc, sgn_ref, bitm_ref):
    acc = xc * (zc[0:1, :] * sgn_ref[0][:, None])
    xp = xc
    for t in range(1, 32):
        j = _GRAY_FLIP[t - 1]
        s = 1 << j
        r_dn = jnp.roll(xp, -s, axis=0)
        r_up = jnp.roll(xp, s, axis=0)
        xp = jnp.where(bitm_ref[j][:, None] != 0.0, r_up, r_dn)
        b = _GRAY[t]
        acc = acc + xp * (zc[b:b + 1, :] * sgn_ref[b][:, None])
    return acc


def _cemlp_block_in_kernel(X, refs, cout, cin, sgb_ref, sgn_ref, bitm_ref):
    a, bias, sa, sb, ag, cn1, cn0 = refs
    h = _mm(a[...], X) + bias[...][:, None]
    q = _mm(sgb_ref[...], h * h)
    gate = jax.nn.sigmoid(sa[...][:, None] * q + sb[...][:, None])
    x2 = h * gate
    z = _mm(ag[...], x2)
    gs = [_gp1(x2[32 * o:32 * (o + 1), :], z[32 * o:32 * (o + 1), :],
               sgn_ref, bitm_ref) for o in range(cout)]
    g = jnp.concatenate(gs, axis=0)
    norm = jnp.sqrt(_mm(sgb_ref[...], g * g) + 1e-6)
    return g / (cn1[...][:, None] * norm + cn0[...][:, None])


def _cemlp_kernel(cin, chid, cout, n_in, x_ref, *refs):
    out_ref = refs[-1]
    sgb_ref, sgn_ref, bitm_ref = refs[-4:-1]
    wrefs = refs[n_in - 1:-4]
    if n_in == 2:
        x2 = x_ref[...] - refs[0][...]
    else:
        x2 = x_ref[...]
    x = x2.T
    x = _cemlp_block_in_kernel(x, wrefs[0:7], chid, cin,
                               sgb_ref, sgn_ref, bitm_ref)
    x = _cemlp_block_in_kernel(x, wrefs[7:14], cout, chid,
                               sgb_ref, sgn_ref, bitm_ref)
    out_ref[...] = x.T


def _expand_linear(w):
    gr = jnp.asarray(_GRADES)
    wexp = w[:, :, gr]
    cout, cin = w.shape[0], w.shape[1]
    eye = jnp.eye(32, dtype=jnp.float32)
    return jnp.einsum('oik,kl->okil', wexp, eye).reshape(cout * 32, cin * 32)


def _prep_block(p):
    gr = jnp.asarray(_GRADES)
    cout = p['W'].shape[0]
    a = _expand_linear(p['W'])
    biasflat = jnp.zeros((cout * 32,), jnp.float32)
    biasflat = biasflat.at[jnp.arange(cout) * 32].set(p['b'])
    saflat = p['sa'][:, gr].reshape(-1)
    sbflat = p['sb'][:, gr].reshape(-1)
    ag = _expand_linear(p['Wg'])
    sig = jax.nn.sigmoid(p['na'])[:, gr].reshape(-1)
    return [a, biasflat, saflat, sbflat, ag, sig, 1.0 - sig]


def _run_cemlp(x_rows, pb0, pb1, cin, chid, cout):
    if not isinstance(x_rows, (list, tuple)):
        x_rows = [x_rows]
    n_in = len(x_rows)
    m = x_rows[0].shape[0]
    grid = m // _BE
    weights = pb0 + pb1 + [jnp.asarray(_SGB), jnp.asarray(_SGN_ROWS),
                           jnp.asarray(_BITM)]
    in_specs = [pl.BlockSpec((_BE, cin * 32), lambda i: (i, 0))
                for _ in range(n_in)]
    for w in weights:
        nd = w.ndim
        in_specs.append(
            pl.BlockSpec(w.shape, functools.partial(lambda n, i: (0,) * n, nd)))
    out_specs = pl.BlockSpec((_BE, cout * 32), lambda i: (i, 0))
    fn = pl.pallas_call(
        functools.partial(_cemlp_kernel, cin, chid, cout, n_in),
        grid=(grid,),
        in_specs=in_specs,
        out_specs=out_specs,
        out_shape=jax.ShapeDtypeStruct((m, cout * 32), jnp.float32),
        compiler_params=pltpu.CompilerParams(
            dimension_semantics=("parallel",)),
    )
    return fn(*x_rows, *weights)


_SC_WORKERS = 32
_GC = 200


def _sc_gather2(h2, src, dst):
    e = src.shape[0]
    d = h2.shape[1]
    per_w = e // _SC_WORKERS
    n_chunks = per_w // _GC
    mesh = plsc.VectorSubcoreMesh(core_axis_name="c", subcore_axis_name="s")
    out_t = (jax.ShapeDtypeStruct((e, d), jnp.float32),
             jax.ShapeDtypeStruct((e, d), jnp.float32))

    @functools.partial(
        pl.kernel, mesh=mesh, out_type=out_t,
        scratch_types=[pltpu.VMEM((_GC,), jnp.int32),
                       pltpu.VMEM((_GC,), jnp.int32),
                       pltpu.VMEM((_GC, d), jnp.float32),
                       pltpu.VMEM((_GC, d), jnp.float32),
                       pltpu.SemaphoreType.DMA,
                       pltpu.SemaphoreType.DMA])
    def k(h_hbm, src_hbm, dst_hbm, od_hbm, os_hbm,
          idx_d, idx_s, rows_d, rows_s, sem_d, sem_s):
        wid = lax.axis_index("s") * 2 + lax.axis_index("c")
        base_w = wid * per_w

        @pl.loop(0, n_chunks)
        def _(j):
            base = base_w + j * _GC
            pltpu.sync_copy(dst_hbm.at[pl.ds(base, _GC)], idx_d)
            pltpu.sync_copy(src_hbm.at[pl.ds(base, _GC)], idx_s)
            cp_d = pltpu.async_copy(h_hbm.at[idx_d], rows_d, sem_d)
            cp_s = pltpu.async_copy(h_hbm.at[idx_s], rows_s, sem_s)
            cp_d.wait()
            cp_s.wait()
            pltpu.sync_copy(rows_d, od_hbm.at[pl.ds(base, _GC)])
            pltpu.sync_copy(rows_s, os_hbm.at[pl.ds(base, _GC)])

    return k(h2, src, dst)


def kernel(input, edge_index, ptr, batch_ids, target, params):
    n_graphs = int(ptr.shape[0]) - 1
    x = input.reshape(n_graphs, -1, 5)
    x = x - x.mean(axis=1, keepdims=True)
    x = x.reshape(-1, 5)
    n = x.shape[0]

    x_mv = jnp.zeros((n, 32), jnp.float32).at[:, _VEC_IDX].set(x)
    h = x_mv[:, None, :] * params['W_embed'][None, :, 0:1]
    h = h.at[:, :, 0].add(params['b_embed'][None, :])
    h2 = h.reshape(n, _HIDDEN * 32)

    src, dst = edge_index[0], edge_index[1]
    e = src.shape[0]

    deg = jax.ops.segment_sum(jnp.ones((e,), jnp.float32), dst, num_segments=n)
    invdeg = 1.0 / jnp.maximum(deg, 1.0)

    n_pad = ((n + _BE - 1) // _BE) * _BE

    for li in range(_N_LAYERS):
        lp = params['layer' + str(li)]
        hd, hs = _sc_gather2(h2, src, dst)
        ep = lp['edge']
        msg = _run_cemlp([hd, hs],
                         _prep_block(ep['b0']), _prep_block(ep['b1']),
                         _HIDDEN, _HIDDEN, _HIDDEN)
        agg = jax.ops.segment_sum(msg, dst, num_segments=n)
        agg = agg * invdeg[:, None]
        node_in = jnp.concatenate([h2, agg], axis=1)
        node_in = jnp.pad(node_in, ((0, n_pad - n), (0, 0)))
        npp = lp['node']
        out2 = _run_cemlp(node_in,
                          _prep_block(npp['b0']), _prep_block(npp['b1']),
                          2 * _HIDDEN, _HIDDEN, _HIDDEN)[:n]
        h2 = h2 + out2

    h_k0 = h2.reshape(n, _HIDDEN, 32)[:, :, 0]
    pred = h_k0 @ params['W_proj'][0, :, 0] + params['b_proj'][0]

    pooled = pred.reshape(n_graphs, n // n_graphs).mean(axis=1)
    loss = (pooled - target) ** 2
    return loss.mean(), loss


# speedup vs baseline: 1.0377x; 1.0377x over previous
import functools

import numpy as np
import jax
import jax.numpy as jnp
from jax import lax
from jax.experimental import pallas as pl
from jax.experimental.pallas import tpu as pltpu
from jax.experimental.pallas import tpu_sc as plsc


def _popcount_py(x):
    return bin(x).count("1")


def _reorder_sign_py(a, b):
    a >>= 1
    s = 0
    while a:
        s += _popcount_py(a & b)
        a >>= 1
    return 1.0 if s % 2 == 0 else -1.0


_GRADES = np.array([_popcount_py(i) for i in range(32)])
_SIGN_K = np.array(
    [[_reorder_sign_py(k ^ b, b) for b in range(32)] for k in range(32)],
    dtype=np.float32)
_VEC_IDX = np.array([1, 2, 4, 8, 16])

_GRAY = [t ^ (t >> 1) for t in range(32)]
_GRAY_FLIP = [int(np.log2(_GRAY[t] ^ _GRAY[t - 1])) for t in range(1, 32)]

_GM = np.stack([( _GRADES == g).astype(np.float32) for g in range(6)])
_SG = (_GRADES[:, None] == _GRADES[None, :]).astype(np.float32)
_SGB = np.kron(np.eye(8, dtype=np.float32), _SG)
_BITM = np.stack([((np.arange(32) >> j) & 1).astype(np.float32)
                  for j in range(5)])
_SGN_ROWS = _SIGN_K.T.copy()

_HIDDEN = 8
_N_LAYERS = 3
_BE = 640


def _mm(a, b, prec=jax.lax.Precision.HIGHEST):
    return jax.lax.dot_general(
        a, b, (((1,), (0,)), ((), ())),
        precision=prec,
        preferred_element_type=jnp.float32)


def _gp1(xc, zc, sgn_ref, bitm_ref):
    acc = xc * (zc[0:1, :] * sgn_ref[0][:, None])
    xp = xc
    for t in range(1, 32):
        j = _GRAY_FLIP[t - 1]
        s = 1 << j
        r_dn = jnp.roll(xp, -s, axis=0)
        r_up = jnp.roll(xp, s, axis=0)
        xp = jnp.where(bitm_ref[j][:, None] != 0.0, r_up, r_dn)
        b = _GRAY[t]
        acc = acc + xp * (zc[b:b + 1, :] * sgn_ref[b][:, None])
    return acc


def _cemlp_block_in_kernel(X, refs, cout, cin, sgb_ref, sgn_ref, bitm_ref):
    a, bias, sa, sb, ag, cn1, cn0 = refs
    h = _mm(a[...], X) + bias[...][:, None]
    q = _mm(sgb_ref[...], h * h, jax.lax.Precision.DEFAULT)
    gate = jax.nn.sigmoid(sa[...][:, None] * q + sb[...][:, None])
    x2 = h * gate
    z = _mm(ag[...], x2)
    gs = [_gp1(x2[32 * o:32 * (o + 1), :], z[32 * o:32 * (o + 1), :],
               sgn_ref, bitm_ref) for o in range(cout)]
    g = jnp.concatenate(gs, axis=0)
    norm = jnp.sqrt(_mm(sgb_ref[...], g * g,
                        jax.lax.Precision.DEFAULT) + 1e-6)
    return g / (cn1[...][:, None] * norm + cn0[...][:, None])


def _cemlp_kernel(cin, chid, cout, n_in, x_ref, *refs):
    out_ref = refs[-1]
    sgb_ref, sgn_ref, bitm_ref = refs[-4:-1]
    wrefs = refs[n_in - 1:-4]
    if n_in == 2:
        x2 = x_ref[...] - refs[0][...]
    else:
        x2 = x_ref[...]
    x = x2.T
    x = _cemlp_block_in_kernel(x, wrefs[0:7], chid, cin,
                               sgb_ref, sgn_ref, bitm_ref)
    x = _cemlp_block_in_kernel(x, wrefs[7:14], cout, chid,
                               sgb_ref, sgn_ref, bitm_ref)
    out_ref[...] = x.T


def _expand_linear(w):
    gr = jnp.asarray(_GRADES)
    wexp = w[:, :, gr]
    cout, cin = w.shape[0], w.shape[1]
    eye = jnp.eye(32, dtype=jnp.float32)
    return jnp.einsum('oik,kl->okil', wexp, eye).reshape(cout * 32, cin * 32)


def _prep_block(p):
    gr = jnp.asarray(_GRADES)
    cout = p['W'].shape[0]
    a = _expand_linear(p['W'])
    biasflat = jnp.zeros((cout * 32,), jnp.float32)
    biasflat = biasflat.at[jnp.arange(cout) * 32].set(p['b'])
    saflat = p['sa'][:, gr].reshape(-1)
    sbflat = p['sb'][:, gr].reshape(-1)
    ag = _expand_linear(p['Wg'])
    sig = jax.nn.sigmoid(p['na'])[:, gr].reshape(-1)
    return [a, biasflat, saflat, sbflat, ag, sig, 1.0 - sig]


def _run_cemlp(x_rows, pb0, pb1, cin, chid, cout):
    if not isinstance(x_rows, (list, tuple)):
        x_rows = [x_rows]
    n_in = len(x_rows)
    m = x_rows[0].shape[0]
    grid = m // _BE
    weights = pb0 + pb1 + [jnp.asarray(_SGB), jnp.asarray(_SGN_ROWS),
                           jnp.asarray(_BITM)]
    in_specs = [pl.BlockSpec((_BE, cin * 32), lambda i: (i, 0))
                for _ in range(n_in)]
    for w in weights:
        nd = w.ndim
        in_specs.append(
            pl.BlockSpec(w.shape, functools.partial(lambda n, i: (0,) * n, nd)))
    out_specs = pl.BlockSpec((_BE, cout * 32), lambda i: (i, 0))
    fn = pl.pallas_call(
        functools.partial(_cemlp_kernel, cin, chid, cout, n_in),
        grid=(grid,),
        in_specs=in_specs,
        out_specs=out_specs,
        out_shape=jax.ShapeDtypeStruct((m, cout * 32), jnp.float32),
        compiler_params=pltpu.CompilerParams(
            dimension_semantics=("parallel",)),
    )
    return fn(*x_rows, *weights)


_SC_WORKERS = 32
_GC = 200


def _sc_gather2(h2, src, dst):
    e = src.shape[0]
    d = h2.shape[1]
    per_w = e // _SC_WORKERS
    n_chunks = per_w // _GC
    mesh = plsc.VectorSubcoreMesh(core_axis_name="c", subcore_axis_name="s")
    out_t = (jax.ShapeDtypeStruct((e, d), jnp.float32),
             jax.ShapeDtypeStruct((e, d), jnp.float32))

    @functools.partial(
        pl.kernel, mesh=mesh, out_type=out_t,
        scratch_types=[pltpu.VMEM((_GC,), jnp.int32),
                       pltpu.VMEM((_GC,), jnp.int32),
                       pltpu.VMEM((_GC, d), jnp.float32),
                       pltpu.VMEM((_GC, d), jnp.float32),
                       pltpu.SemaphoreType.DMA,
                       pltpu.SemaphoreType.DMA])
    def k(h_hbm, src_hbm, dst_hbm, od_hbm, os_hbm,
          idx_d, idx_s, rows_d, rows_s, sem_d, sem_s):
        wid = lax.axis_index("s") * 2 + lax.axis_index("c")
        base_w = wid * per_w

        @pl.loop(0, n_chunks)
        def _(j):
            base = base_w + j * _GC
            pltpu.sync_copy(dst_hbm.at[pl.ds(base, _GC)], idx_d)
            pltpu.sync_copy(src_hbm.at[pl.ds(base, _GC)], idx_s)
            cp_d = pltpu.async_copy(h_hbm.at[idx_d], rows_d, sem_d)
            cp_s = pltpu.async_copy(h_hbm.at[idx_s], rows_s, sem_s)
            cp_d.wait()
            cp_s.wait()
            pltpu.sync_copy(rows_d, od_hbm.at[pl.ds(base, _GC)])
            pltpu.sync_copy(rows_s, os_hbm.at[pl.ds(base, _GC)])

    return k(h2, src, dst)


def kernel(input, edge_index, ptr, batch_ids, target, params):
    n_graphs = int(ptr.shape[0]) - 1
    x = input.reshape(n_graphs, -1, 5)
    x = x - x.mean(axis=1, keepdims=True)
    x = x.reshape(-1, 5)
    n = x.shape[0]

    x_mv = jnp.zeros((n, 32), jnp.float32).at[:, _VEC_IDX].set(x)
    h = x_mv[:, None, :] * params['W_embed'][None, :, 0:1]
    h = h.at[:, :, 0].add(params['b_embed'][None, :])
    h2 = h.reshape(n, _HIDDEN * 32)

    src, dst = edge_index[0], edge_index[1]
    e = src.shape[0]

    deg = jax.ops.segment_sum(jnp.ones((e,), jnp.float32), dst, num_segments=n)
    invdeg = 1.0 / jnp.maximum(deg, 1.0)

    n_pad = ((n + _BE - 1) // _BE) * _BE

    for li in range(_N_LAYERS):
        lp = params['layer' + str(li)]
        hd, hs = _sc_gather2(h2, src, dst)
        ep = lp['edge']
        msg = _run_cemlp([hd, hs],
                         _prep_block(ep['b0']), _prep_block(ep['b1']),
                         _HIDDEN, _HIDDEN, _HIDDEN)
        agg = jax.ops.segment_sum(msg, dst, num_segments=n)
        agg = agg * invdeg[:, None]
        node_in = jnp.concatenate([h2, agg], axis=1)
        node_in = jnp.pad(node_in, ((0, n_pad - n), (0, 0)))
        npp = lp['node']
        out2 = _run_cemlp(node_in,
                          _prep_block(npp['b0']), _prep_block(npp['b1']),
                          2 * _HIDDEN, _HIDDEN, _HIDDEN)[:n]
        h2 = h2 + out2

    h_k0 = h2.reshape(n, _HIDDEN, 32)[:, :, 0]
    pred = h_k0 @ params['W_proj'][0, :, 0] + params['b_proj'][0]

    pooled = pred.reshape(n_graphs, n // n_graphs).mean(axis=1)
    loss = (pooled - target) ** 2
    return loss.mean(), loss


# speedup vs baseline: 1.1203x; 1.0796x over previous
import functools

import numpy as np
import jax
import jax.numpy as jnp
from jax import lax
from jax.experimental import pallas as pl
from jax.experimental.pallas import tpu as pltpu
from jax.experimental.pallas import tpu_sc as plsc


def _popcount_py(x):
    return bin(x).count("1")


def _reorder_sign_py(a, b):
    a >>= 1
    s = 0
    while a:
        s += _popcount_py(a & b)
        a >>= 1
    return 1.0 if s % 2 == 0 else -1.0


_GRADES = np.array([_popcount_py(i) for i in range(32)])
_SIGN_K = np.array(
    [[_reorder_sign_py(k ^ b, b) for b in range(32)] for k in range(32)],
    dtype=np.float32)
_VEC_IDX = np.array([1, 2, 4, 8, 16])

_GRAY = [t ^ (t >> 1) for t in range(32)]
_GRAY_FLIP = [int(np.log2(_GRAY[t] ^ _GRAY[t - 1])) for t in range(1, 32)]

_GM = np.stack([( _GRADES == g).astype(np.float32) for g in range(6)])
_SG = (_GRADES[:, None] == _GRADES[None, :]).astype(np.float32)
_SGB = np.kron(np.eye(8, dtype=np.float32), _SG)
_BITM = np.stack([((np.arange(32) >> j) & 1).astype(np.float32)
                  for j in range(5)])
_SGN_ROWS = _SIGN_K.T.copy()

_HIDDEN = 8
_N_LAYERS = 3
_BE = 640


def _mm(a, b, prec=jax.lax.Precision.HIGHEST):
    return jax.lax.dot_general(
        a, b, (((1,), (0,)), ((), ())),
        precision=prec,
        preferred_element_type=jnp.float32)


def _gp1(xc, zc, sgn_ref, bitm_ref):
    acc = xc * (zc[0:1, :] * sgn_ref[0][:, None])
    xp = xc
    for t in range(1, 32):
        j = _GRAY_FLIP[t - 1]
        s = 1 << j
        r_dn = jnp.roll(xp, -s, axis=0)
        r_up = jnp.roll(xp, s, axis=0)
        xp = jnp.where(bitm_ref[j][:, None] != 0.0, r_up, r_dn)
        b = _GRAY[t]
        acc = acc + xp * (zc[b:b + 1, :] * sgn_ref[b][:, None])
    return acc


def _cemlp_block_in_kernel(X, refs, cout, cin, sgb_ref, sgn_ref, bitm_ref):
    a, bias, sa, sb, ag, cn1, cn0 = refs
    h = _mm(a[...], X) + bias[...][:, None]
    q = _mm(sgb_ref[...], h * h, jax.lax.Precision.DEFAULT)
    gate = jax.nn.sigmoid(sa[...][:, None] * q + sb[...][:, None])
    x2 = h * gate
    z = _mm(ag[...], x2)
    gs = [_gp1(x2[32 * o:32 * (o + 1), :], z[32 * o:32 * (o + 1), :],
               sgn_ref, bitm_ref) for o in range(cout)]
    g = jnp.concatenate(gs, axis=0)
    norm = jnp.sqrt(_mm(sgb_ref[...], g * g,
                        jax.lax.Precision.DEFAULT) + 1e-6)
    return g / (cn1[...][:, None] * norm + cn0[...][:, None])


def _cemlp_kernel(cin, chid, cout, n_in, x_ref, *refs):
    out_ref = refs[-1]
    sgb_ref, sgn_ref, bitm_ref = refs[-4:-1]
    wrefs = refs[n_in - 1:-4]
    if n_in == 2:
        x2 = x_ref[...] - refs[0][...]
    else:
        x2 = x_ref[...]
    x = x2.T
    x = _cemlp_block_in_kernel(x, wrefs[0:7], chid, cin,
                               sgb_ref, sgn_ref, bitm_ref)
    x = _cemlp_block_in_kernel(x, wrefs[7:14], cout, chid,
                               sgb_ref, sgn_ref, bitm_ref)
    out_ref[...] = x.T


def _expand_linear(w):
    gr = jnp.asarray(_GRADES)
    wexp = w[:, :, gr]
    cout, cin = w.shape[0], w.shape[1]
    eye = jnp.eye(32, dtype=jnp.float32)
    return jnp.einsum('oik,kl->okil', wexp, eye).reshape(cout * 32, cin * 32)


def _prep_block(p):
    gr = jnp.asarray(_GRADES)
    cout = p['W'].shape[0]
    a = _expand_linear(p['W'])
    biasflat = jnp.zeros((cout * 32,), jnp.float32)
    biasflat = biasflat.at[jnp.arange(cout) * 32].set(p['b'])
    saflat = p['sa'][:, gr].reshape(-1)
    sbflat = p['sb'][:, gr].reshape(-1)
    ag = _expand_linear(p['Wg'])
    sig = jax.nn.sigmoid(p['na'])[:, gr].reshape(-1)
    return [a, biasflat, saflat, sbflat, ag, sig, 1.0 - sig]


def _run_cemlp(x_rows, pb0, pb1, cin, chid, cout):
    if not isinstance(x_rows, (list, tuple)):
        x_rows = [x_rows]
    n_in = len(x_rows)
    m = x_rows[0].shape[0]
    grid = m // _BE
    weights = pb0 + pb1 + [jnp.asarray(_SGB), jnp.asarray(_SGN_ROWS),
                           jnp.asarray(_BITM)]
    in_specs = [pl.BlockSpec((_BE, cin * 32), lambda i: (i, 0))
                for _ in range(n_in)]
    for w in weights:
        nd = w.ndim
        in_specs.append(
            pl.BlockSpec(w.shape, functools.partial(lambda n, i: (0,) * n, nd)))
    out_specs = pl.BlockSpec((_BE, cout * 32), lambda i: (i, 0))
    fn = pl.pallas_call(
        functools.partial(_cemlp_kernel, cin, chid, cout, n_in),
        grid=(grid,),
        in_specs=in_specs,
        out_specs=out_specs,
        out_shape=jax.ShapeDtypeStruct((m, cout * 32), jnp.float32),
        compiler_params=pltpu.CompilerParams(
            dimension_semantics=("parallel",)),
    )
    return fn(*x_rows, *weights)


_SC_WORKERS = 32


def _pick_gc(per_w):
    for c in range(200, 0, -8):
        if per_w % c == 0:
            return c
    raise ValueError(per_w)


def _sc_gather2(h2, src, dst):
    e = src.shape[0]
    d = h2.shape[1]
    per_w = e // _SC_WORKERS
    _GC = _pick_gc(per_w)
    n_chunks = per_w // _GC
    mesh = plsc.VectorSubcoreMesh(core_axis_name="c", subcore_axis_name="s")
    out_t = (jax.ShapeDtypeStruct((e, d), jnp.float32),
             jax.ShapeDtypeStruct((e, d), jnp.float32))

    @functools.partial(
        pl.kernel, mesh=mesh, out_type=out_t,
        scratch_types=[pltpu.VMEM((_GC,), jnp.int32),
                       pltpu.VMEM((_GC,), jnp.int32),
                       pltpu.VMEM((_GC, d), jnp.float32),
                       pltpu.VMEM((_GC, d), jnp.float32),
                       pltpu.SemaphoreType.DMA,
                       pltpu.SemaphoreType.DMA])
    def k(h_hbm, src_hbm, dst_hbm, od_hbm, os_hbm,
          idx_d, idx_s, rows_d, rows_s, sem_d, sem_s):
        wid = lax.axis_index("s") * 2 + lax.axis_index("c")
        base_w = wid * per_w

        @pl.loop(0, n_chunks)
        def _(j):
            base = base_w + j * _GC
            pltpu.sync_copy(dst_hbm.at[pl.ds(base, _GC)], idx_d)
            pltpu.sync_copy(src_hbm.at[pl.ds(base, _GC)], idx_s)
            cp_d = pltpu.async_copy(h_hbm.at[idx_d], rows_d, sem_d)
            cp_s = pltpu.async_copy(h_hbm.at[idx_s], rows_s, sem_s)
            cp_d.wait()
            cp_s.wait()
            pltpu.sync_copy(rows_d, od_hbm.at[pl.ds(base, _GC)])
            pltpu.sync_copy(rows_s, os_hbm.at[pl.ds(base, _GC)])

    return k(h2, src, dst)


def kernel(input, edge_index, ptr, batch_ids, target, params):
    n_graphs = int(ptr.shape[0]) - 1
    x = input.reshape(n_graphs, -1, 5)
    x = x - x.mean(axis=1, keepdims=True)
    x = x.reshape(-1, 5)
    n = x.shape[0]

    x_mv = jnp.zeros((n, 32), jnp.float32).at[:, _VEC_IDX].set(x)
    h = x_mv[:, None, :] * params['W_embed'][None, :, 0:1]
    h = h.at[:, :, 0].add(params['b_embed'][None, :])
    h2 = h.reshape(n, _HIDDEN * 32)

    src, dst = edge_index[0], edge_index[1]
    e = src.shape[0]

    deg = jax.ops.segment_sum(jnp.ones((e,), jnp.float32), dst, num_segments=n)
    invdeg = 1.0 / jnp.maximum(deg, 1.0)

    n_pad = ((n + _BE - 1) // _BE) * _BE

    half = (e // 2 // 1280) * 1280
    bounds = [(0, half), (half, e)]

    for li in range(_N_LAYERS):
        lp = params['layer' + str(li)]
        ep = lp['edge']
        pb0, pb1 = _prep_block(ep['b0']), _prep_block(ep['b1'])
        agg = None
        for lo, hi in bounds:
            dst_c = dst[lo:hi]
            hd, hs = _sc_gather2(h2, src[lo:hi], dst_c)
            msg = _run_cemlp([hd, hs], pb0, pb1,
                             _HIDDEN, _HIDDEN, _HIDDEN)
            part = jax.ops.segment_sum(msg, dst_c, num_segments=n)
            agg = part if agg is None else agg + part
        agg = agg * invdeg[:, None]
        node_in = jnp.concatenate([h2, agg], axis=1)
        node_in = jnp.pad(node_in, ((0, n_pad - n), (0, 0)))
        npp = lp['node']
        out2 = _run_cemlp(node_in,
                          _prep_block(npp['b0']), _prep_block(npp['b1']),
                          2 * _HIDDEN, _HIDDEN, _HIDDEN)[:n]
        h2 = h2 + out2

    h_k0 = h2.reshape(n, _HIDDEN, 32)[:, :, 0]
    pred = h_k0 @ params['W_proj'][0, :, 0] + params['b_proj'][0]

    pooled = pred.reshape(n_graphs, n // n_graphs).mean(axis=1)
    loss = (pooled - target) ** 2
    return loss.mean(), loss


# speedup vs baseline: 1.1615x; 1.0367x over previous
import functools

import numpy as np
import jax
import jax.numpy as jnp
from jax import lax
from jax.experimental import pallas as pl
from jax.experimental.pallas import tpu as pltpu
from jax.experimental.pallas import tpu_sc as plsc


def _popcount_py(x):
    return bin(x).count("1")


def _reorder_sign_py(a, b):
    a >>= 1
    s = 0
    while a:
        s += _popcount_py(a & b)
        a >>= 1
    return 1.0 if s % 2 == 0 else -1.0


_GRADES = np.array([_popcount_py(i) for i in range(32)])
_SIGN_K = np.array(
    [[_reorder_sign_py(k ^ b, b) for b in range(32)] for k in range(32)],
    dtype=np.float32)
_VEC_IDX = np.array([1, 2, 4, 8, 16])

_GRAY = [t ^ (t >> 1) for t in range(32)]
_GRAY_FLIP = [int(np.log2(_GRAY[t] ^ _GRAY[t - 1])) for t in range(1, 32)]

_GM = np.stack([( _GRADES == g).astype(np.float32) for g in range(6)])
_SG = (_GRADES[:, None] == _GRADES[None, :]).astype(np.float32)
_SGB = np.kron(np.eye(8, dtype=np.float32), _SG)
_BITM = np.stack([((np.arange(32) >> j) & 1).astype(np.float32)
                  for j in range(5)])
_SGN_ROWS = _SIGN_K.T.copy()

_HIDDEN = 8
_N_LAYERS = 3
_BE = 640


def _mm(a, b, prec=jax.lax.Precision.HIGHEST):
    return jax.lax.dot_general(
        a, b, (((1,), (0,)), ((), ())),
        precision=prec,
        preferred_element_type=jnp.float32)


def _gp1(xc, zc, sgn_ref, bitm_ref):
    acc = xc * (zc[0:1, :] * sgn_ref[0][:, None])
    xp = xc
    for t in range(1, 32):
        j = _GRAY_FLIP[t - 1]
        s = 1 << j
        r_dn = jnp.roll(xp, -s, axis=0)
        r_up = jnp.roll(xp, s, axis=0)
        xp = jnp.where(bitm_ref[j][:, None] != 0.0, r_up, r_dn)
        b = _GRAY[t]
        acc = acc + xp * (zc[b:b + 1, :] * sgn_ref[b][:, None])
    return acc


def _cemlp_block_in_kernel(X, refs, cout, cin, sgb_ref, sgn_ref, bitm_ref):
    a, bias, sa, sb, ag, cn1, cn0 = refs
    h = _mm(a[...], X) + bias[...][:, None]
    q = _mm(sgb_ref[...], h * h, jax.lax.Precision.DEFAULT)
    gate = jax.nn.sigmoid(sa[...][:, None] * q + sb[...][:, None])
    x2 = h * gate
    z = _mm(ag[...], x2)
    gs = [_gp1(x2[32 * o:32 * (o + 1), :], z[32 * o:32 * (o + 1), :],
               sgn_ref, bitm_ref) for o in range(cout)]
    g = jnp.concatenate(gs, axis=0)
    norm = jnp.sqrt(_mm(sgb_ref[...], g * g,
                        jax.lax.Precision.DEFAULT) + 1e-6)
    return g / (cn1[...][:, None] * norm + cn0[...][:, None])


def _cemlp_kernel(cin, chid, cout, n_in, x_ref, *refs):
    out_ref = refs[-1]
    sgb_ref, sgn_ref, bitm_ref = refs[-4:-1]
    wrefs = refs[n_in - 1:-4]
    if n_in == 2:
        x2 = x_ref[...] - refs[0][...]
    else:
        x2 = x_ref[...]
    x = x2.T
    x = _cemlp_block_in_kernel(x, wrefs[0:7], chid, cin,
                               sgb_ref, sgn_ref, bitm_ref)
    x = _cemlp_block_in_kernel(x, wrefs[7:14], cout, chid,
                               sgb_ref, sgn_ref, bitm_ref)
    out_ref[...] = x.T


def _expand_linear(w):
    gr = jnp.asarray(_GRADES)
    wexp = w[:, :, gr]
    cout, cin = w.shape[0], w.shape[1]
    eye = jnp.eye(32, dtype=jnp.float32)
    return jnp.einsum('oik,kl->okil', wexp, eye).reshape(cout * 32, cin * 32)


def _prep_block(p):
    gr = jnp.asarray(_GRADES)
    cout = p['W'].shape[0]
    a = _expand_linear(p['W'])
    biasflat = jnp.zeros((cout * 32,), jnp.float32)
    biasflat = biasflat.at[jnp.arange(cout) * 32].set(p['b'])
    saflat = p['sa'][:, gr].reshape(-1)
    sbflat = p['sb'][:, gr].reshape(-1)
    ag = _expand_linear(p['Wg'])
    sig = jax.nn.sigmoid(p['na'])[:, gr].reshape(-1)
    return [a, biasflat, saflat, sbflat, ag, sig, 1.0 - sig]


def _run_cemlp(x_rows, pb0, pb1, cin, chid, cout):
    if not isinstance(x_rows, (list, tuple)):
        x_rows = [x_rows]
    n_in = len(x_rows)
    m = x_rows[0].shape[0]
    grid = m // _BE
    weights = pb0 + pb1 + [jnp.asarray(_SGB), jnp.asarray(_SGN_ROWS),
                           jnp.asarray(_BITM)]
    in_specs = [pl.BlockSpec((_BE, cin * 32), lambda i: (i, 0))
                for _ in range(n_in)]
    for w in weights:
        nd = w.ndim
        in_specs.append(
            pl.BlockSpec(w.shape, functools.partial(lambda n, i: (0,) * n, nd)))
    out_specs = pl.BlockSpec((_BE, cout * 32), lambda i: (i, 0))
    fn = pl.pallas_call(
        functools.partial(_cemlp_kernel, cin, chid, cout, n_in),
        grid=(grid,),
        in_specs=in_specs,
        out_specs=out_specs,
        out_shape=jax.ShapeDtypeStruct((m, cout * 32), jnp.float32),
        compiler_params=pltpu.CompilerParams(
            dimension_semantics=("parallel",)),
    )
    return fn(*x_rows, *weights)


_SC_WORKERS = 32


def _pick_gc(per_w):
    for c in range(200, 0, -8):
        if per_w % c == 0:
            return c
    raise ValueError(per_w)


def _sc_gather2(h2, src, dst):
    e = src.shape[0]
    d = h2.shape[1]
    per_w = e // _SC_WORKERS
    _GC = _pick_gc(per_w)
    n_chunks = per_w // _GC
    mesh = plsc.VectorSubcoreMesh(core_axis_name="c", subcore_axis_name="s")
    out_t = (jax.ShapeDtypeStruct((e, d), jnp.float32),
             jax.ShapeDtypeStruct((e, d), jnp.float32))

    @functools.partial(
        pl.kernel, mesh=mesh, out_type=out_t,
        scratch_types=[pltpu.VMEM((_GC,), jnp.int32),
                       pltpu.VMEM((_GC,), jnp.int32),
                       pltpu.VMEM((_GC, d), jnp.float32),
                       pltpu.VMEM((_GC, d), jnp.float32),
                       pltpu.SemaphoreType.DMA,
                       pltpu.SemaphoreType.DMA])
    def k(h_hbm, src_hbm, dst_hbm, od_hbm, os_hbm,
          idx_d, idx_s, rows_d, rows_s, sem_d, sem_s):
        wid = lax.axis_index("s") * 2 + lax.axis_index("c")
        base_w = wid * per_w

        @pl.loop(0, n_chunks)
        def _(j):
            base = base_w + j * _GC
            pltpu.sync_copy(dst_hbm.at[pl.ds(base, _GC)], idx_d)
            pltpu.sync_copy(src_hbm.at[pl.ds(base, _GC)], idx_s)
            cp_d = pltpu.async_copy(h_hbm.at[idx_d], rows_d, sem_d)
            cp_s = pltpu.async_copy(h_hbm.at[idx_s], rows_s, sem_s)
            cp_d.wait()
            cp_s.wait()
            pltpu.sync_copy(rows_d, od_hbm.at[pl.ds(base, _GC)])
            pltpu.sync_copy(rows_s, os_hbm.at[pl.ds(base, _GC)])

    return k(h2, src, dst)


def kernel(input, edge_index, ptr, batch_ids, target, params):
    n_graphs = int(ptr.shape[0]) - 1
    x = input.reshape(n_graphs, -1, 5)
    x = x - x.mean(axis=1, keepdims=True)
    x = x.reshape(-1, 5)
    n = x.shape[0]

    x_mv = jnp.zeros((n, 32), jnp.float32).at[:, _VEC_IDX].set(x)
    h = x_mv[:, None, :] * params['W_embed'][None, :, 0:1]
    h = h.at[:, :, 0].add(params['b_embed'][None, :])
    h2 = h.reshape(n, _HIDDEN * 32)

    src, dst = edge_index[0], edge_index[1]
    e = src.shape[0]

    deg = jax.ops.segment_sum(jnp.ones((e,), jnp.float32), dst, num_segments=n)
    invdeg = 1.0 / jnp.maximum(deg, 1.0)

    n_pad = ((n + _BE - 1) // _BE) * _BE

    n_chunk = 4
    unit = 1280
    units = e // unit
    sizes = [(units // n_chunk + (1 if ci < units % n_chunk else 0)) * unit
             for ci in range(n_chunk)]
    sizes[-1] += e - units * unit
    bounds = []
    lo = 0
    for sz in sizes:
        if sz:
            bounds.append((lo, lo + sz))
            lo += sz

    for li in range(_N_LAYERS):
        lp = params['layer' + str(li)]
        ep = lp['edge']
        pb0, pb1 = _prep_block(ep['b0']), _prep_block(ep['b1'])
        agg = None
        for lo, hi in bounds:
            dst_c = dst[lo:hi]
            hd, hs = _sc_gather2(h2, src[lo:hi], dst_c)
            msg = _run_cemlp([hd, hs], pb0, pb1,
                             _HIDDEN, _HIDDEN, _HIDDEN)
            part = jax.ops.segment_sum(msg, dst_c, num_segments=n)
            agg = part if agg is None else agg + part
        agg = agg * invdeg[:, None]
        node_in = jnp.concatenate([h2, agg], axis=1)
        node_in = jnp.pad(node_in, ((0, n_pad - n), (0, 0)))
        npp = lp['node']
        out2 = _run_cemlp(node_in,
                          _prep_block(npp['b0']), _prep_block(npp['b1']),
                          2 * _HIDDEN, _HIDDEN, _HIDDEN)[:n]
        h2 = h2 + out2

    h_k0 = h2.reshape(n, _HIDDEN, 32)[:, :, 0]
    pred = h_k0 @ params['W_proj'][0, :, 0] + params['b_proj'][0]

    pooled = pred.reshape(n_graphs, n // n_graphs).mean(axis=1)
    loss = (pooled - target) ** 2
    return loss.mean(), loss


# speedup vs baseline: 1.1918x; 1.0260x over previous
import functools

import numpy as np
import jax
import jax.numpy as jnp
from jax import lax
from jax.experimental import pallas as pl
from jax.experimental.pallas import tpu as pltpu
from jax.experimental.pallas import tpu_sc as plsc


def _popcount_py(x):
    return bin(x).count("1")


def _reorder_sign_py(a, b):
    a >>= 1
    s = 0
    while a:
        s += _popcount_py(a & b)
        a >>= 1
    return 1.0 if s % 2 == 0 else -1.0


_GRADES = np.array([_popcount_py(i) for i in range(32)])
_SIGN_K = np.array(
    [[_reorder_sign_py(k ^ b, b) for b in range(32)] for k in range(32)],
    dtype=np.float32)
_VEC_IDX = np.array([1, 2, 4, 8, 16])

_GRAY = [t ^ (t >> 1) for t in range(32)]
_GRAY_FLIP = [int(np.log2(_GRAY[t] ^ _GRAY[t - 1])) for t in range(1, 32)]

_GM = np.stack([( _GRADES == g).astype(np.float32) for g in range(6)])
_SG = (_GRADES[:, None] == _GRADES[None, :]).astype(np.float32)
_SGB = np.kron(np.eye(8, dtype=np.float32), _SG)
_BITM = np.stack([((np.arange(32) >> j) & 1).astype(np.float32)
                  for j in range(5)])
_SGN_ROWS = _SIGN_K.T.copy()

_HIDDEN = 8
_N_LAYERS = 3
_BE = 640


def _mm(a, b, prec=jax.lax.Precision.HIGHEST):
    return jax.lax.dot_general(
        a, b, (((1,), (0,)), ((), ())),
        precision=prec,
        preferred_element_type=jnp.float32)


def _gp1(xc, zc, sgn_ref, bitm_ref):
    acc = xc * (zc[0:1, :] * sgn_ref[0][:, None])
    xp = xc
    for t in range(1, 32):
        j = _GRAY_FLIP[t - 1]
        s = 1 << j
        r_dn = jnp.roll(xp, -s, axis=0)
        r_up = jnp.roll(xp, s, axis=0)
        xp = jnp.where(bitm_ref[j][:, None] != 0.0, r_up, r_dn)
        b = _GRAY[t]
        acc = acc + xp * (zc[b:b + 1, :] * sgn_ref[b][:, None])
    return acc


def _cemlp_block_in_kernel(X, refs, cout, cin, sgb_ref, sgn_ref, bitm_ref):
    a, bias, sa, sb, ag, cn1, cn0 = refs
    h = _mm(a[...], X) + bias[...][:, None]
    q = _mm(sgb_ref[...], h * h, jax.lax.Precision.DEFAULT)
    gate = jax.nn.sigmoid(sa[...][:, None] * q + sb[...][:, None])
    x2 = h * gate
    z = _mm(ag[...], x2)
    gs = [_gp1(x2[32 * o:32 * (o + 1), :], z[32 * o:32 * (o + 1), :],
               sgn_ref, bitm_ref) for o in range(cout)]
    g = jnp.concatenate(gs, axis=0)
    norm = jnp.sqrt(_mm(sgb_ref[...], g * g,
                        jax.lax.Precision.DEFAULT) + 1e-6)
    return g / (cn1[...][:, None] * norm + cn0[...][:, None])


def _cemlp_kernel(cin, chid, cout, n_in, x_ref, *refs):
    out_ref = refs[-1]
    sgb_ref, sgn_ref, bitm_ref = refs[-4:-1]
    wrefs = refs[n_in - 1:-4]
    if n_in == 2:
        x2 = x_ref[...] - refs[0][...]
    else:
        x2 = x_ref[...]
    x = x2.T
    x = _cemlp_block_in_kernel(x, wrefs[0:7], chid, cin,
                               sgb_ref, sgn_ref, bitm_ref)
    x = _cemlp_block_in_kernel(x, wrefs[7:14], cout, chid,
                               sgb_ref, sgn_ref, bitm_ref)
    out_ref[...] = x.T


def _expand_linear(w):
    gr = jnp.asarray(_GRADES)
    wexp = w[:, :, gr]
    cout, cin = w.shape[0], w.shape[1]
    eye = jnp.eye(32, dtype=jnp.float32)
    return jnp.einsum('oik,kl->okil', wexp, eye).reshape(cout * 32, cin * 32)


def _prep_block(p):
    gr = jnp.asarray(_GRADES)
    cout = p['W'].shape[0]
    a = _expand_linear(p['W'])
    biasflat = jnp.zeros((cout * 32,), jnp.float32)
    biasflat = biasflat.at[jnp.arange(cout) * 32].set(p['b'])
    saflat = p['sa'][:, gr].reshape(-1)
    sbflat = p['sb'][:, gr].reshape(-1)
    ag = _expand_linear(p['Wg'])
    sig = jax.nn.sigmoid(p['na'])[:, gr].reshape(-1)
    return [a, biasflat, saflat, sbflat, ag, sig, 1.0 - sig]


def _run_cemlp(x_rows, pb0, pb1, cin, chid, cout):
    if not isinstance(x_rows, (list, tuple)):
        x_rows = [x_rows]
    n_in = len(x_rows)
    m = x_rows[0].shape[0]
    grid = m // _BE
    weights = pb0 + pb1 + [jnp.asarray(_SGB), jnp.asarray(_SGN_ROWS),
                           jnp.asarray(_BITM)]
    in_specs = [pl.BlockSpec((_BE, cin * 32), lambda i: (i, 0))
                for _ in range(n_in)]
    for w in weights:
        nd = w.ndim
        in_specs.append(
            pl.BlockSpec(w.shape, functools.partial(lambda n, i: (0,) * n, nd)))
    out_specs = pl.BlockSpec((_BE, cout * 32), lambda i: (i, 0))
    fn = pl.pallas_call(
        functools.partial(_cemlp_kernel, cin, chid, cout, n_in),
        grid=(grid,),
        in_specs=in_specs,
        out_specs=out_specs,
        out_shape=jax.ShapeDtypeStruct((m, cout * 32), jnp.float32),
        compiler_params=pltpu.CompilerParams(
            dimension_semantics=("parallel",)),
    )
    return fn(*x_rows, *weights)


_SC_WORKERS = 32


def _pick_gc(per_w):
    for c in range(200, 0, -8):
        if per_w % c == 0:
            return c
    raise ValueError(per_w)


def _sc_gather2(h2, src, dst):
    e = src.shape[0]
    d = h2.shape[1]
    per_w = e // _SC_WORKERS
    _GC = _pick_gc(per_w)
    n_chunks = per_w // _GC
    mesh = plsc.VectorSubcoreMesh(core_axis_name="c", subcore_axis_name="s")
    out_t = (jax.ShapeDtypeStruct((e, d), jnp.float32),
             jax.ShapeDtypeStruct((e, d), jnp.float32))

    @functools.partial(
        pl.kernel, mesh=mesh, out_type=out_t,
        scratch_types=[pltpu.VMEM((_GC,), jnp.int32),
                       pltpu.VMEM((_GC,), jnp.int32),
                       pltpu.VMEM((_GC, d), jnp.float32),
                       pltpu.VMEM((_GC, d), jnp.float32),
                       pltpu.SemaphoreType.DMA,
                       pltpu.SemaphoreType.DMA])
    def k(h_hbm, src_hbm, dst_hbm, od_hbm, os_hbm,
          idx_d, idx_s, rows_d, rows_s, sem_d, sem_s):
        wid = lax.axis_index("s") * 2 + lax.axis_index("c")
        base_w = wid * per_w

        @pl.loop(0, n_chunks)
        def _(j):
            base = base_w + j * _GC
            pltpu.sync_copy(dst_hbm.at[pl.ds(base, _GC)], idx_d)
            pltpu.sync_copy(src_hbm.at[pl.ds(base, _GC)], idx_s)
            cp_d = pltpu.async_copy(h_hbm.at[idx_d], rows_d, sem_d)
            cp_s = pltpu.async_copy(h_hbm.at[idx_s], rows_s, sem_s)
            cp_d.wait()
            cp_s.wait()
            pltpu.sync_copy(rows_d, od_hbm.at[pl.ds(base, _GC)])
            pltpu.sync_copy(rows_s, os_hbm.at[pl.ds(base, _GC)])

    return k(h2, src, dst)


def kernel(input, edge_index, ptr, batch_ids, target, params):
    n_graphs = int(ptr.shape[0]) - 1
    x = input.reshape(n_graphs, -1, 5)
    x = x - x.mean(axis=1, keepdims=True)
    x = x.reshape(-1, 5)
    n = x.shape[0]

    x_mv = jnp.zeros((n, 32), jnp.float32).at[:, _VEC_IDX].set(x)
    h = x_mv[:, None, :] * params['W_embed'][None, :, 0:1]
    h = h.at[:, :, 0].add(params['b_embed'][None, :])
    h2 = h.reshape(n, _HIDDEN * 32)

    src, dst = edge_index[0], edge_index[1]
    e = src.shape[0]

    deg = jax.ops.segment_sum(jnp.ones((e,), jnp.float32), dst, num_segments=n)
    invdeg = 1.0 / jnp.maximum(deg, 1.0)

    n_pad = ((n + _BE - 1) // _BE) * _BE

    n_chunk = 8
    unit = 1280
    units = e // unit
    sizes = [(units // n_chunk + (1 if ci < units % n_chunk else 0)) * unit
             for ci in range(n_chunk)]
    sizes[-1] += e - units * unit
    bounds = []
    lo = 0
    for sz in sizes:
        if sz:
            bounds.append((lo, lo + sz))
            lo += sz

    for li in range(_N_LAYERS):
        lp = params['layer' + str(li)]
        ep = lp['edge']
        pb0, pb1 = _prep_block(ep['b0']), _prep_block(ep['b1'])
        agg = None
        for lo, hi in bounds:
            dst_c = dst[lo:hi]
            hd, hs = _sc_gather2(h2, src[lo:hi], dst_c)
            msg = _run_cemlp([hd, hs], pb0, pb1,
                             _HIDDEN, _HIDDEN, _HIDDEN)
            part = jax.ops.segment_sum(msg, dst_c, num_segments=n)
            agg = part if agg is None else agg + part
        agg = agg * invdeg[:, None]
        node_in = jnp.concatenate([h2, agg], axis=1)
        node_in = jnp.pad(node_in, ((0, n_pad - n), (0, 0)))
        npp = lp['node']
        out2 = _run_cemlp(node_in,
                          _prep_block(npp['b0']), _prep_block(npp['b1']),
                          2 * _HIDDEN, _HIDDEN, _HIDDEN)[:n]
        h2 = h2 + out2

    h_k0 = h2.reshape(n, _HIDDEN, 32)[:, :, 0]
    pred = h_k0 @ params['W_proj'][0, :, 0] + params['b_proj'][0]

    pooled = pred.reshape(n_graphs, n // n_graphs).mean(axis=1)
    loss = (pooled - target) ** 2
    return loss.mean(), loss
